# baseline (speedup 1.0000x reference)
import functools

import jax
import jax.numpy as jnp
from jax import lax
from jax.experimental import pallas as pl
from jax.experimental.pallas import tpu as pltpu
from jax.experimental.pallas import tpu_sc as plsc

EMBED_DIM = 32
BATCH = 16384
_NUM_CORES = 2
_NUM_SUBCORES = 16
_NW = _NUM_CORES * _NUM_SUBCORES
_BPW = BATCH // _NW


@functools.partial(
    pl.kernel,
    mesh=plsc.VectorSubcoreMesh(core_axis_name="c", subcore_axis_name="s"),
    out_type=(
        jax.ShapeDtypeStruct((BATCH, EMBED_DIM), jnp.float32),
        jax.ShapeDtypeStruct((BATCH,), jnp.float32),
    ),
    scratch_types=[
        pltpu.VMEM((_BPW,), jnp.int32),
        pltpu.VMEM((_BPW, EMBED_DIM), jnp.float32),
        pltpu.VMEM((_BPW,), jnp.float32),
        pltpu.SemaphoreType.DMA,
        pltpu.SemaphoreType.DMA,
    ],
    compiler_params=pltpu.CompilerParams(use_tc_tiling_on_sc=False),
)
def _sc_gather(idx_hbm, table_hbm, bias_hbm, vec_out, bias_out,
               idx_v, rows_v, bias_v, sem_rows, sem_bias):
    wid = lax.axis_index("s") * _NUM_CORES + lax.axis_index("c")
    base = wid * _BPW
    pltpu.sync_copy(idx_hbm.at[pl.ds(base, _BPW)], idx_v)
    cp_rows = pltpu.async_copy(table_hbm.at[idx_v], rows_v, sem_rows)
    cp_bias = pltpu.async_copy(bias_hbm.at[idx_v], bias_v, sem_bias)
    cp_rows.wait()
    pltpu.sync_copy(rows_v, vec_out.at[pl.ds(base, _BPW)])
    cp_bias.wait()
    pltpu.sync_copy(bias_v, bias_out.at[pl.ds(base, _BPW)])


def kernel(user_id, emb_users, bias_user):
    user_vec, user_bias = _sc_gather(user_id, emb_users,
                                     bias_user.reshape(-1))
    return (user_vec, user_bias)


# speedup vs baseline: 2.7322x; 2.7322x over previous
import functools

import jax
import jax.numpy as jnp
from jax import lax
from jax.experimental import pallas as pl
from jax.experimental.pallas import tpu as pltpu
from jax.experimental.pallas import tpu_sc as plsc

EMBED_DIM = 32
BATCH = 16384
_NUM_CORES = 2
_NUM_SUBCORES = 16
_NW = _NUM_CORES * _NUM_SUBCORES
_BPW = BATCH // _NW
_G = 16


@functools.partial(
    pl.kernel,
    mesh=plsc.VectorSubcoreMesh(core_axis_name="c", subcore_axis_name="s"),
    out_type=(
        jax.ShapeDtypeStruct((EMBED_DIM, BATCH), jnp.float32),
        jax.ShapeDtypeStruct((BATCH,), jnp.float32),
    ),
    scratch_types=[
        pltpu.VMEM((_BPW,), jnp.int32),
        pltpu.VMEM((_G, EMBED_DIM, 128), jnp.float32),
        pltpu.VMEM((EMBED_DIM * _BPW,), jnp.float32),
        pltpu.VMEM((_BPW,), jnp.float32),
        pltpu.SemaphoreType.DMA,
        pltpu.SemaphoreType.DMA,
    ],
    compiler_params=pltpu.CompilerParams(needs_layout_passes=False),
)
def _sc_gather(idx_hbm, embt_hbm, bias_hbm, vect_out, bias_out,
               idx_v, ring, cols_v, bias_v, sem_rows, sem_bias):
    wid = lax.axis_index("s") * _NUM_CORES + lax.axis_index("c")
    base = wid * _BPW
    pltpu.sync_copy(idx_hbm.at[pl.ds(base, _BPW)], idx_v)

    cp_bias = pltpu.async_copy(bias_hbm.at[idx_v], bias_v, sem_bias)

    iota = lax.iota(jnp.int32, 16)

    def _group(g, _):
        v16 = idx_v[pl.ds(g * _G, _G)]
        for j in range(_G):
            r = v16[j]
            start = pl.multiple_of(
                lax.shift_left(lax.shift_right_logical(r, 7), 7), 128)
            pltpu.async_copy(embt_hbm.at[:, pl.ds(start, 128)],
                             ring.at[j], sem_rows)
        for j in range(_G):
            pltpu.make_async_copy(embt_hbm.at[:, pl.ds(0, 128)],
                                  ring.at[j], sem_rows).wait()
        lanes = v16 & 127
        for j in range(_G):
            lane = jnp.broadcast_to(lanes[j], (16,))
            slot = g * _G + j
            for h in range(EMBED_DIM // 16):
                c16 = iota + h * 16
                vals = plsc.load_gather(ring.at[j], [c16, lane])
                plsc.store_scatter(cols_v, [c16 * _BPW + slot], vals)
        return 0

    lax.fori_loop(0, _BPW // _G, _group, 0)

    for c in range(EMBED_DIM):
        pltpu.sync_copy(cols_v.at[pl.ds(c * _BPW, _BPW)],
                        vect_out.at[c, pl.ds(base, _BPW)])
    cp_bias.wait()
    pltpu.sync_copy(bias_v, bias_out.at[pl.ds(base, _BPW)])


def kernel(user_id, emb_users, bias_user):
    vec_t, user_bias = _sc_gather(user_id, emb_users.T,
                                  bias_user.reshape(-1))
    return (vec_t.T, user_bias)


# speedup vs baseline: 3.1456x; 1.1513x over previous
import functools

import jax
import jax.numpy as jnp
from jax import lax
from jax.experimental import pallas as pl
from jax.experimental.pallas import tpu as pltpu
from jax.experimental.pallas import tpu_sc as plsc

EMBED_DIM = 32
BATCH = 16384
_NUM_CORES = 2
_NUM_SUBCORES = 16
_NW = _NUM_CORES * _NUM_SUBCORES
_BPW = BATCH // _NW
_G = 16
_NG = _BPW // _G


@functools.partial(
    pl.kernel,
    mesh=plsc.VectorSubcoreMesh(core_axis_name="c", subcore_axis_name="s"),
    out_type=(
        jax.ShapeDtypeStruct((EMBED_DIM, BATCH), jnp.float32),
        jax.ShapeDtypeStruct((BATCH,), jnp.float32),
    ),
    scratch_types=[
        pltpu.VMEM((_BPW,), jnp.int32),
        pltpu.VMEM((_G, EMBED_DIM, 128), jnp.float32),
        pltpu.VMEM((EMBED_DIM, _BPW), jnp.float32),
        pltpu.VMEM((_BPW,), jnp.float32),
        pltpu.SemaphoreType.DMA,
    ] + [pltpu.SemaphoreType.DMA] * _G,
    compiler_params=pltpu.CompilerParams(needs_layout_passes=False),
)
def _sc_gather(idx_hbm, embt_hbm, bias_hbm, vect_out, bias_out,
               idx_v, ring, cols2, bias_v, sem_bias, *sem_rows):
    wid = lax.axis_index("s") * _NUM_CORES + lax.axis_index("c")
    base = wid * _BPW
    pltpu.sync_copy(idx_hbm.at[pl.ds(base, _BPW)], idx_v)

    cp_bias = pltpu.async_copy(bias_hbm.at[idx_v], bias_v, sem_bias)

    iota = lax.iota(jnp.int32, 16)

    def _fire(v16, j):
        start = pl.multiple_of(
            lax.shift_left(lax.shift_right_logical(v16[j], 7), 7), 128)
        pltpu.async_copy(embt_hbm.at[:, pl.ds(start, 128)],
                         ring.at[j], sem_rows[j])

    def _extract(v16, g, j):
        lane = jnp.broadcast_to(v16[j] & 127, (16,))
        slot = jnp.broadcast_to(g * _G + j, (16,))
        for h in range(EMBED_DIM // 16):
            c16 = iota + h * 16
            vals = plsc.load_gather(ring.at[j], [c16, lane])
            plsc.store_scatter(cols2, [c16, slot], vals)

    v16_0 = idx_v[pl.ds(0, _G)]
    for j in range(_G):
        _fire(v16_0, j)

    def _steady(g, v_prev):
        v_next = idx_v[pl.ds(g * _G, _G)]
        for j in range(_G):
            pltpu.make_async_copy(embt_hbm.at[:, pl.ds(0, 128)],
                                  ring.at[j], sem_rows[j]).wait()
            _extract(v_prev, g - 1, j)
            _fire(v_next, j)
        return v_next

    v_last = lax.fori_loop(1, _NG, _steady, v16_0)

    for j in range(_G):
        pltpu.make_async_copy(embt_hbm.at[:, pl.ds(0, 128)],
                              ring.at[j], sem_rows[j]).wait()
        _extract(v_last, _NG - 1, j)

    pltpu.sync_copy(cols2, vect_out.at[:, pl.ds(base, _BPW)])
    cp_bias.wait()
    pltpu.sync_copy(bias_v, bias_out.at[pl.ds(base, _BPW)])


def kernel(user_id, emb_users, bias_user):
    vec_t, user_bias = _sc_gather(user_id, emb_users.T,
                                  bias_user.reshape(-1))
    return (vec_t.T, user_bias)
